# baseline (device time: 91033 ns/iter reference)
import jax
import jax.numpy as jnp
from jax import lax
from jax.experimental import pallas as pl
from jax.experimental.pallas import tpu as pltpu

N_DEV = 4
SQ = 2048
D_MODEL = 1024
H_LOC = 8
DH = 128
NR = 4
NB = SQ // (NR * 64)
RG = SQ // NR
SCALE = 0.08838834764831843

OFFSETS = (2, 1, 3, 0)


def _gather(dst_ref, src_ref, c):
    for b in range(NB):
        dst_ref[pl.ds(b * 64, 64)] = src_ref[pl.ds(b * NR * 64 + c * 64, 64)]


def _scatter_out(out_ref, val, c):
    for b in range(NB):
        out_ref[pl.ds(b * NR * 64 + c * 64, 64), :] = val[b * 64:(b + 1) * 64, :]


def _body(x_ref, wq_ref, k_ref, v_ref, wo_ref, out_ref,
          xc_ref, kc_ref, vc_ref, own_ref,
          scat_ref, rs_recv_ref, bcast_ref, ag_recv_ref,
          p1_send, p1_recv, p2_send, p2_recv, ctx_ref):
    my = lax.axis_index("i")

    barrier_sem = pltpu.get_barrier_semaphore()
    for off in (1, 2, 3):
        peer = lax.rem(my + off, N_DEV)
        pl.semaphore_signal(
            barrier_sem, inc=1,
            device_id=(peer,), device_id_type=pl.DeviceIdType.MESH,
        )
    pl.semaphore_wait(barrier_sem, 3)

    p1_rdmas = []
    for off in OFFSETS:
        c = lax.rem(my + off, N_DEV)
        _gather(xc_ref, x_ref, c)
        _gather(kc_ref, k_ref, c)
        _gather(vc_ref, v_ref, c)
        q = jnp.dot(xc_ref[...], wq_ref[...],
                    preferred_element_type=jnp.float32)
        q = q.astype(jnp.bfloat16)
        kc = kc_ref[...]
        vc = vc_ref[...]
        for h in range(H_LOC):
            qh = q[:, h * DH:(h + 1) * DH]
            scores = lax.dot_general(
                qh, kc[:, h, :],
                (((1,), (1,)), ((), ())),
                preferred_element_type=jnp.float32,
            )
            w = jnp.exp(scores)
            w = (w / jnp.sum(w, axis=1, keepdims=True)).astype(jnp.bfloat16)
            ctx_ref[:, h * DH:(h + 1) * DH] = jnp.dot(
                w, vc[:, h, :], preferred_element_type=jnp.float32
            ).astype(jnp.bfloat16)
        partial = jnp.dot(ctx_ref[...], wo_ref[...],
                          preferred_element_type=jnp.float32)
        if off == 0:
            own_ref[...] = partial
        else:
            slot = 3 - off
            scat_ref[slot] = partial.astype(jnp.bfloat16)
            rdma = pltpu.make_async_remote_copy(
                src_ref=scat_ref.at[slot],
                dst_ref=rs_recv_ref.at[slot],
                send_sem=p1_send.at[slot],
                recv_sem=p1_recv.at[slot],
                device_id=(c,),
                device_id_type=pl.DeviceIdType.MESH,
            )
            rdma.start()
            p1_rdmas.append(rdma)

    red = own_ref[...]
    for s in range(3):
        recv = pltpu.make_async_remote_copy(
            src_ref=rs_recv_ref.at[s],
            dst_ref=rs_recv_ref.at[s],
            send_sem=p1_send.at[s],
            recv_sem=p1_recv.at[s],
            device_id=(my,),
            device_id_type=pl.DeviceIdType.MESH,
        )
        recv.wait_recv()
        red = red + rs_recv_ref[s].astype(jnp.float32)
    bcast_ref[...] = red.astype(jnp.bfloat16)

    p2_rdmas = []
    for off in (1, 2, 3):
        peer = lax.rem(my + off, N_DEV)
        slot = 3 - off
        rdma = pltpu.make_async_remote_copy(
            src_ref=bcast_ref,
            dst_ref=ag_recv_ref.at[slot],
            send_sem=p2_send.at[slot],
            recv_sem=p2_recv.at[slot],
            device_id=(peer,),
            device_id_type=pl.DeviceIdType.MESH,
        )
        rdma.start()
        p2_rdmas.append(rdma)

    _scatter_out(out_ref, red, my)

    for s in (0, 2, 1):
        recv = pltpu.make_async_remote_copy(
            src_ref=ag_recv_ref.at[s],
            dst_ref=ag_recv_ref.at[s],
            send_sem=p2_send.at[s],
            recv_sem=p2_recv.at[s],
            device_id=(my,),
            device_id_type=pl.DeviceIdType.MESH,
        )
        recv.wait_recv()
        chunk = lax.rem(my + s + 1, N_DEV)
        _scatter_out(out_ref, ag_recv_ref[s].astype(jnp.float32), chunk)

    for rdma in p1_rdmas + p2_rdmas:
        rdma.wait_send()


def kernel(x, Wq, K_ext, V_ext, Wo):
    i = lax.axis_index("i")

    x_b = x[0].astype(jnp.bfloat16)
    K_b = lax.dynamic_slice_in_dim(
        K_ext[0], i * H_LOC, H_LOC, axis=1).astype(jnp.bfloat16)
    V_b = lax.dynamic_slice_in_dim(
        V_ext[0], i * H_LOC, H_LOC, axis=1).astype(jnp.bfloat16)
    Wq_b = (Wq * SCALE).astype(jnp.bfloat16)
    Wo_b = Wo.astype(jnp.bfloat16)

    out = pl.pallas_call(
        _body,
        out_shape=jax.ShapeDtypeStruct((SQ, D_MODEL), jnp.float32),
        in_specs=[pl.BlockSpec(memory_space=pltpu.VMEM)] * 5,
        out_specs=pl.BlockSpec(memory_space=pltpu.VMEM),
        scratch_shapes=[
            pltpu.VMEM((RG, D_MODEL), jnp.bfloat16),
            pltpu.VMEM((RG, H_LOC, DH), jnp.bfloat16),
            pltpu.VMEM((RG, H_LOC, DH), jnp.bfloat16),
            pltpu.VMEM((RG, D_MODEL), jnp.float32),
            pltpu.VMEM((3, RG, D_MODEL), jnp.bfloat16),
            pltpu.VMEM((3, RG, D_MODEL), jnp.bfloat16),
            pltpu.VMEM((RG, D_MODEL), jnp.bfloat16),
            pltpu.VMEM((3, RG, D_MODEL), jnp.bfloat16),
            pltpu.SemaphoreType.DMA((3,)),
            pltpu.SemaphoreType.DMA((3,)),
            pltpu.SemaphoreType.DMA((3,)),
            pltpu.SemaphoreType.DMA((3,)),
            pltpu.VMEM((RG, H_LOC * DH), jnp.bfloat16),
        ],
        compiler_params=pltpu.CompilerParams(collective_id=0),
    )(x_b, Wq_b, K_b, V_b, Wo_b)

    return out[None]


# device time: 80666 ns/iter; 1.1285x vs baseline; 1.1285x over previous
import jax
import jax.numpy as jnp
from jax import lax
from jax.experimental import pallas as pl
from jax.experimental.pallas import tpu as pltpu

N_DEV = 4
SQ = 2048
D_MODEL = 1024
H_LOC = 8
DH = 128
NR = 4
NB = SQ // (NR * 64)
RG = SQ // NR
SCALE = 0.08838834764831843

OFFSETS = (2, 1, 3, 0)


def _gather(dst_ref, src_ref, c):
    for b in range(NB):
        dst_ref[pl.ds(b * 64, 64)] = (
            src_ref[pl.ds(b * NR * 64 + c * 64, 64)].astype(dst_ref.dtype))


def _scatter_out(out_ref, val, c):
    for b in range(NB):
        out_ref[pl.ds(b * NR * 64 + c * 64, 64), :] = val[b * 64:(b + 1) * 64, :]


def _body(x_ref, wq_ref, k_ref, v_ref, wo_ref, out_ref,
          xc_ref, kc_ref, vc_ref, own_ref, wqb_ref, wob_ref,
          scat_ref, rs_recv_ref, bcast_ref, ag_recv_ref,
          p1_send, p1_recv, p2_send, p2_recv, ctx_ref):
    my = lax.axis_index("i")

    barrier_sem = pltpu.get_barrier_semaphore()
    for off in (1, 2, 3):
        peer = lax.rem(my + off, N_DEV)
        pl.semaphore_signal(
            barrier_sem, inc=1,
            device_id=(peer,), device_id_type=pl.DeviceIdType.MESH,
        )
    pl.semaphore_wait(barrier_sem, 3)

    wqb_ref[...] = (wq_ref[...] * SCALE).astype(jnp.bfloat16)
    wob_ref[...] = wo_ref[...].astype(jnp.bfloat16)

    p1_rdmas = []
    for off in OFFSETS:
        c = lax.rem(my + off, N_DEV)
        _gather(xc_ref, x_ref, c)
        _gather(kc_ref, k_ref, c)
        _gather(vc_ref, v_ref, c)
        q = jnp.dot(xc_ref[...], wqb_ref[...],
                    preferred_element_type=jnp.float32)
        q = q.astype(jnp.bfloat16)
        kc = kc_ref[...]
        vc = vc_ref[...]
        for h in range(H_LOC):
            qh = q[:, h * DH:(h + 1) * DH]
            scores = lax.dot_general(
                qh, kc[:, h, :],
                (((1,), (1,)), ((), ())),
                preferred_element_type=jnp.float32,
            )
            w = jnp.exp(scores)
            w = (w / jnp.sum(w, axis=1, keepdims=True)).astype(jnp.bfloat16)
            ctx_ref[:, h * DH:(h + 1) * DH] = jnp.dot(
                w, vc[:, h, :], preferred_element_type=jnp.float32
            ).astype(jnp.bfloat16)
        partial = jnp.dot(ctx_ref[...], wob_ref[...],
                          preferred_element_type=jnp.float32)
        if off == 0:
            own_ref[...] = partial
        else:
            slot = 3 - off
            scat_ref[slot] = partial.astype(jnp.bfloat16)
            rdma = pltpu.make_async_remote_copy(
                src_ref=scat_ref.at[slot],
                dst_ref=rs_recv_ref.at[slot],
                send_sem=p1_send.at[slot],
                recv_sem=p1_recv.at[slot],
                device_id=(c,),
                device_id_type=pl.DeviceIdType.MESH,
            )
            rdma.start()
            p1_rdmas.append(rdma)

    red = own_ref[...]
    for s in range(3):
        recv = pltpu.make_async_remote_copy(
            src_ref=rs_recv_ref.at[s],
            dst_ref=rs_recv_ref.at[s],
            send_sem=p1_send.at[s],
            recv_sem=p1_recv.at[s],
            device_id=(my,),
            device_id_type=pl.DeviceIdType.MESH,
        )
        recv.wait_recv()
        red = red + rs_recv_ref[s].astype(jnp.float32)
    red_b = red.astype(jnp.bfloat16)
    bcast_ref[...] = red_b

    p2_rdmas = []
    for off in (1, 2, 3):
        peer = lax.rem(my + off, N_DEV)
        slot = 3 - off
        rdma = pltpu.make_async_remote_copy(
            src_ref=bcast_ref,
            dst_ref=ag_recv_ref.at[slot],
            send_sem=p2_send.at[slot],
            recv_sem=p2_recv.at[slot],
            device_id=(peer,),
            device_id_type=pl.DeviceIdType.MESH,
        )
        rdma.start()
        p2_rdmas.append(rdma)

    _scatter_out(out_ref, red_b, my)

    for s in (0, 2, 1):
        recv = pltpu.make_async_remote_copy(
            src_ref=ag_recv_ref.at[s],
            dst_ref=ag_recv_ref.at[s],
            send_sem=p2_send.at[s],
            recv_sem=p2_recv.at[s],
            device_id=(my,),
            device_id_type=pl.DeviceIdType.MESH,
        )
        recv.wait_recv()
        chunk = lax.rem(my + s + 1, N_DEV)
        _scatter_out(out_ref, ag_recv_ref[s], chunk)

    for rdma in p1_rdmas + p2_rdmas:
        rdma.wait_send()


def kernel(x, Wq, K_ext, V_ext, Wo):
    i = lax.axis_index("i")

    x_b = x[0]
    K_b = lax.dynamic_slice_in_dim(
        K_ext[0], i * H_LOC, H_LOC, axis=1).astype(jnp.bfloat16)
    V_b = lax.dynamic_slice_in_dim(
        V_ext[0], i * H_LOC, H_LOC, axis=1).astype(jnp.bfloat16)

    out = pl.pallas_call(
        _body,
        out_shape=jax.ShapeDtypeStruct((SQ, D_MODEL), jnp.bfloat16),
        in_specs=[pl.BlockSpec(memory_space=pltpu.VMEM)] * 5,
        out_specs=pl.BlockSpec(memory_space=pltpu.VMEM),
        scratch_shapes=[
            pltpu.VMEM((RG, D_MODEL), jnp.bfloat16),
            pltpu.VMEM((RG, H_LOC, DH), jnp.bfloat16),
            pltpu.VMEM((RG, H_LOC, DH), jnp.bfloat16),
            pltpu.VMEM((RG, D_MODEL), jnp.float32),
            pltpu.VMEM((D_MODEL, D_MODEL), jnp.bfloat16),
            pltpu.VMEM((D_MODEL, D_MODEL), jnp.bfloat16),
            pltpu.VMEM((3, RG, D_MODEL), jnp.bfloat16),
            pltpu.VMEM((3, RG, D_MODEL), jnp.bfloat16),
            pltpu.VMEM((RG, D_MODEL), jnp.bfloat16),
            pltpu.VMEM((3, RG, D_MODEL), jnp.bfloat16),
            pltpu.SemaphoreType.DMA((3,)),
            pltpu.SemaphoreType.DMA((3,)),
            pltpu.SemaphoreType.DMA((3,)),
            pltpu.SemaphoreType.DMA((3,)),
            pltpu.VMEM((RG, H_LOC * DH), jnp.bfloat16),
        ],
        compiler_params=pltpu.CompilerParams(collective_id=0),
    )(x_b, Wq, K_b, V_b, Wo)

    return out[None]


# device time: 73384 ns/iter; 1.2405x vs baseline; 1.0992x over previous
import jax
import jax.numpy as jnp
from jax import lax
from jax.experimental import pallas as pl
from jax.experimental.pallas import tpu as pltpu

N_DEV = 4
SQ = 2048
D_MODEL = 1024
H_LOC = 8
DH = 128
NR = 4
NB = SQ // (NR * 64)
RG = SQ // NR
HG = RG // 2
SCALE = 0.08838834764831843

OFFSETS = (2, 1, 3, 0)


def _gather(dst_ref, src_ref, c):
    for b in range(NB):
        dst_ref[pl.ds(b * 64, 64)] = (
            src_ref[pl.ds(b * NR * 64 + c * 64, 64)].astype(dst_ref.dtype))


def _scatter_out_half(out_ref, val, c, half):
    for b in range(NB // 2):
        bb = half * (NB // 2) + b
        out_ref[pl.ds(bb * NR * 64 + c * 64, 64), :] = (
            val[b * 64:(b + 1) * 64, :])


def _body(x_ref, wq_ref, k_ref, v_ref, wo_ref, out_ref,
          xc_ref, kf_ref, vf_ref, kc_ref, vc_ref, own_ref, wqb_ref, wob_ref,
          scat_ref, rs_recv_ref, bcast_ref, ag_recv_ref,
          kv_sem, p1_send, p1_recv, p2_send, p2_recv, ctx_ref):
    my = lax.axis_index("i")

    def prefetch_kv(c, buf):
        copies = []
        for b in range(NB):
            row = b * NR * 64 + c * 64
            for src, dst in ((k_ref, kf_ref), (v_ref, vf_ref)):
                cp = pltpu.make_async_copy(
                    src.at[pl.ds(row, 64), pl.ds(my * H_LOC, H_LOC), :],
                    dst.at[buf, pl.ds(b * 64, 64)],
                    kv_sem.at[buf],
                )
                cp.start()
                copies.append(cp)
        return copies

    pending = prefetch_kv(lax.rem(my + OFFSETS[0], N_DEV), 0)

    barrier_sem = pltpu.get_barrier_semaphore()
    for off in (1, 2, 3):
        peer = lax.rem(my + off, N_DEV)
        pl.semaphore_signal(
            barrier_sem, inc=1,
            device_id=(peer,), device_id_type=pl.DeviceIdType.MESH,
        )
    pl.semaphore_wait(barrier_sem, 3)

    wqb_ref[...] = (wq_ref[...] * SCALE).astype(jnp.bfloat16)
    wob_ref[...] = wo_ref[...].astype(jnp.bfloat16)

    p1_rdmas = []
    for idx, off in enumerate(OFFSETS):
        c = lax.rem(my + off, N_DEV)
        buf = idx % 2
        if idx + 1 < N_DEV:
            nxt = prefetch_kv(lax.rem(my + OFFSETS[idx + 1], N_DEV), 1 - buf)
        for cp in pending:
            cp.wait()
        pending = nxt if idx + 1 < N_DEV else []
        kc_ref[...] = kf_ref[buf].astype(jnp.bfloat16)
        vc_ref[...] = vf_ref[buf].astype(jnp.bfloat16)
        _gather(xc_ref, x_ref, c)
        q = jnp.dot(xc_ref[...], wqb_ref[...],
                    preferred_element_type=jnp.float32)
        q = q.astype(jnp.bfloat16)
        kc = kc_ref[...]
        vc = vc_ref[...]
        for h in range(H_LOC):
            qh = q[:, h * DH:(h + 1) * DH]
            scores = lax.dot_general(
                qh, kc[:, h, :],
                (((1,), (1,)), ((), ())),
                preferred_element_type=jnp.float32,
            )
            w = jnp.exp(scores)
            w = (w / jnp.sum(w, axis=1, keepdims=True)).astype(jnp.bfloat16)
            ctx_ref[:, h * DH:(h + 1) * DH] = jnp.dot(
                w, vc[:, h, :], preferred_element_type=jnp.float32
            ).astype(jnp.bfloat16)
        partial = jnp.dot(ctx_ref[...], wob_ref[...],
                          preferred_element_type=jnp.float32)
        if off == 0:
            own_ref[...] = partial
        else:
            slot = 3 - off
            scat_ref[slot] = partial.astype(jnp.bfloat16)
            for half in range(2):
                rdma = pltpu.make_async_remote_copy(
                    src_ref=scat_ref.at[slot, pl.ds(half * HG, HG)],
                    dst_ref=rs_recv_ref.at[slot, pl.ds(half * HG, HG)],
                    send_sem=p1_send.at[slot * 2 + half],
                    recv_sem=p1_recv.at[slot * 2 + half],
                    device_id=(c,),
                    device_id_type=pl.DeviceIdType.MESH,
                )
                rdma.start()
                p1_rdmas.append(rdma)

    p2_rdmas = []
    for half in range(2):
        rows = pl.ds(half * HG, HG)
        red = own_ref[rows, :]
        for s in range(3):
            recv = pltpu.make_async_remote_copy(
                src_ref=rs_recv_ref.at[s, rows],
                dst_ref=rs_recv_ref.at[s, rows],
                send_sem=p1_send.at[s * 2 + half],
                recv_sem=p1_recv.at[s * 2 + half],
                device_id=(my,),
                device_id_type=pl.DeviceIdType.MESH,
            )
            recv.wait_recv()
            red = red + rs_recv_ref[s, rows, :].astype(jnp.float32)
        red_b = red.astype(jnp.bfloat16)
        bcast_ref[rows, :] = red_b
        for off in (1, 2, 3):
            peer = lax.rem(my + off, N_DEV)
            slot = 3 - off
            rdma = pltpu.make_async_remote_copy(
                src_ref=bcast_ref.at[rows],
                dst_ref=ag_recv_ref.at[slot, rows],
                send_sem=p2_send.at[slot * 2 + half],
                recv_sem=p2_recv.at[slot * 2 + half],
                device_id=(peer,),
                device_id_type=pl.DeviceIdType.MESH,
            )
            rdma.start()
            p2_rdmas.append(rdma)
        _scatter_out_half(out_ref, red_b, my, half)

    for half in range(2):
        for s in (0, 2, 1):
            rows = pl.ds(half * HG, HG)
            recv = pltpu.make_async_remote_copy(
                src_ref=ag_recv_ref.at[s, rows],
                dst_ref=ag_recv_ref.at[s, rows],
                send_sem=p2_send.at[s * 2 + half],
                recv_sem=p2_recv.at[s * 2 + half],
                device_id=(my,),
                device_id_type=pl.DeviceIdType.MESH,
            )
            recv.wait_recv()
            chunk = lax.rem(my + s + 1, N_DEV)
            _scatter_out_half(out_ref, ag_recv_ref[s, half * HG:(half + 1) * HG, :],
                              chunk, half)

    for rdma in p1_rdmas + p2_rdmas:
        rdma.wait_send()


def kernel(x, Wq, K_ext, V_ext, Wo):
    out = pl.pallas_call(
        _body,
        out_shape=jax.ShapeDtypeStruct((SQ, D_MODEL), jnp.bfloat16),
        in_specs=[
            pl.BlockSpec(memory_space=pltpu.VMEM),
            pl.BlockSpec(memory_space=pltpu.VMEM),
            pl.BlockSpec(memory_space=pl.ANY),
            pl.BlockSpec(memory_space=pl.ANY),
            pl.BlockSpec(memory_space=pltpu.VMEM),
        ],
        out_specs=pl.BlockSpec(memory_space=pltpu.VMEM),
        scratch_shapes=[
            pltpu.VMEM((RG, D_MODEL), jnp.bfloat16),
            pltpu.VMEM((2, RG, H_LOC, DH), jnp.float32),
            pltpu.VMEM((2, RG, H_LOC, DH), jnp.float32),
            pltpu.VMEM((RG, H_LOC, DH), jnp.bfloat16),
            pltpu.VMEM((RG, H_LOC, DH), jnp.bfloat16),
            pltpu.VMEM((RG, D_MODEL), jnp.float32),
            pltpu.VMEM((D_MODEL, D_MODEL), jnp.bfloat16),
            pltpu.VMEM((D_MODEL, D_MODEL), jnp.bfloat16),
            pltpu.VMEM((3, RG, D_MODEL), jnp.bfloat16),
            pltpu.VMEM((3, RG, D_MODEL), jnp.bfloat16),
            pltpu.VMEM((RG, D_MODEL), jnp.bfloat16),
            pltpu.VMEM((3, RG, D_MODEL), jnp.bfloat16),
            pltpu.SemaphoreType.DMA((2,)),
            pltpu.SemaphoreType.DMA((6,)),
            pltpu.SemaphoreType.DMA((6,)),
            pltpu.SemaphoreType.DMA((6,)),
            pltpu.SemaphoreType.DMA((6,)),
            pltpu.VMEM((RG, H_LOC * DH), jnp.bfloat16),
        ],
        compiler_params=pltpu.CompilerParams(
            collective_id=0, vmem_limit_bytes=100 * 1024 * 1024),
    )(x[0], Wq, K_ext[0], V_ext[0], Wo)

    return out[None]


# device time: 71532 ns/iter; 1.2726x vs baseline; 1.0259x over previous
import jax
import jax.numpy as jnp
from jax import lax
from jax.experimental import pallas as pl
from jax.experimental.pallas import tpu as pltpu

N_DEV = 4
SQ = 2048
D_MODEL = 1024
H_LOC = 8
DH = 128
NR = 4
NB = SQ // (NR * 64)
RG = SQ // NR
HG = RG // 2
SCALE = 0.08838834764831843

OFFSETS = (2, 1, 3, 0)


def _gather(dst_ref, src_ref, c):
    for b in range(NB):
        dst_ref[pl.ds(b * 64, 64)] = (
            src_ref[pl.ds(b * NR * 64 + c * 64, 64)].astype(dst_ref.dtype))


def _scatter_out_half(out_ref, val, c, half):
    for b in range(NB // 2):
        bb = half * (NB // 2) + b
        out_ref[pl.ds(bb * NR * 64 + c * 64, 64), :] = (
            val[b * 64:(b + 1) * 64, :])


def _body(x_ref, wq_ref, k_ref, v_ref, wo_ref, out_ref,
          xf_ref, wqf_ref, wof_ref, kf_ref, vf_ref, xc_ref, kc_ref, vc_ref,
          own_ref, wqb_ref, wob_ref,
          scat_ref, rs_recv_ref, bcast_ref, ag_recv_ref,
          kv_sem, w_sem, p1_send, p1_recv, p2_send, p2_recv, ctx_ref):
    my = lax.axis_index("i")

    def prefetch_chunk(c, buf):
        copies = []
        for b in range(NB):
            row = b * NR * 64 + c * 64
            cp = pltpu.make_async_copy(
                x_ref.at[pl.ds(row, 64)],
                xf_ref.at[buf, pl.ds(b * 64, 64)],
                kv_sem.at[buf],
            )
            cp.start()
            copies.append(cp)
            for src, dst in ((k_ref, kf_ref), (v_ref, vf_ref)):
                cp = pltpu.make_async_copy(
                    src.at[pl.ds(row, 64), pl.ds(my * H_LOC, H_LOC), :],
                    dst.at[buf, pl.ds(b * 64, 64)],
                    kv_sem.at[buf],
                )
                cp.start()
                copies.append(cp)
        return copies

    w_copies = [
        pltpu.make_async_copy(wq_ref, wqf_ref, w_sem),
        pltpu.make_async_copy(wo_ref, wof_ref, w_sem),
    ]
    for cp in w_copies:
        cp.start()
    pending = prefetch_chunk(lax.rem(my + OFFSETS[0], N_DEV), 0)

    barrier_sem = pltpu.get_barrier_semaphore()
    for off in (1, 2, 3):
        peer = lax.rem(my + off, N_DEV)
        pl.semaphore_signal(
            barrier_sem, inc=1,
            device_id=(peer,), device_id_type=pl.DeviceIdType.MESH,
        )
    pl.semaphore_wait(barrier_sem, 3)

    for cp in w_copies:
        cp.wait()
    wqb_ref[...] = (wqf_ref[...] * SCALE).astype(jnp.bfloat16)
    wob_ref[...] = wof_ref[...].astype(jnp.bfloat16)

    p1_rdmas = []
    for idx, off in enumerate(OFFSETS):
        c = lax.rem(my + off, N_DEV)
        buf = idx % 2
        if idx + 1 < N_DEV:
            nxt = prefetch_chunk(lax.rem(my + OFFSETS[idx + 1], N_DEV), 1 - buf)
        for cp in pending:
            cp.wait()
        pending = nxt if idx + 1 < N_DEV else []
        kc_ref[...] = kf_ref[buf].astype(jnp.bfloat16)
        vc_ref[...] = vf_ref[buf].astype(jnp.bfloat16)
        xc_ref[...] = xf_ref[buf].astype(jnp.bfloat16)
        q = jnp.dot(xc_ref[...], wqb_ref[...],
                    preferred_element_type=jnp.float32)
        q = q.astype(jnp.bfloat16)
        kc = kc_ref[...]
        vc = vc_ref[...]
        for h in range(H_LOC):
            qh = q[:, h * DH:(h + 1) * DH]
            scores = lax.dot_general(
                qh, kc[:, h, :],
                (((1,), (1,)), ((), ())),
                preferred_element_type=jnp.float32,
            )
            w = jnp.exp(scores)
            w = (w / jnp.sum(w, axis=1, keepdims=True)).astype(jnp.bfloat16)
            ctx_ref[:, h * DH:(h + 1) * DH] = jnp.dot(
                w, vc[:, h, :], preferred_element_type=jnp.float32
            ).astype(jnp.bfloat16)
        partial = jnp.dot(ctx_ref[...], wob_ref[...],
                          preferred_element_type=jnp.float32)
        if off == 0:
            own_ref[...] = partial
        else:
            slot = 3 - off
            scat_ref[slot] = partial.astype(jnp.bfloat16)
            for half in range(2):
                rdma = pltpu.make_async_remote_copy(
                    src_ref=scat_ref.at[slot, pl.ds(half * HG, HG)],
                    dst_ref=rs_recv_ref.at[slot, pl.ds(half * HG, HG)],
                    send_sem=p1_send.at[slot * 2 + half],
                    recv_sem=p1_recv.at[slot * 2 + half],
                    device_id=(c,),
                    device_id_type=pl.DeviceIdType.MESH,
                )
                rdma.start()
                p1_rdmas.append(rdma)

    p2_rdmas = []
    for half in range(2):
        rows = pl.ds(half * HG, HG)
        red = own_ref[rows, :]
        for s in (1, 2, 0):
            recv = pltpu.make_async_remote_copy(
                src_ref=rs_recv_ref.at[s, rows],
                dst_ref=rs_recv_ref.at[s, rows],
                send_sem=p1_send.at[s * 2 + half],
                recv_sem=p1_recv.at[s * 2 + half],
                device_id=(my,),
                device_id_type=pl.DeviceIdType.MESH,
            )
            recv.wait_recv()
            red = red + rs_recv_ref[s, rows, :].astype(jnp.float32)
        red_b = red.astype(jnp.bfloat16)
        bcast_ref[rows, :] = red_b
        for off in (1, 2, 3):
            peer = lax.rem(my + off, N_DEV)
            slot = 3 - off
            rdma = pltpu.make_async_remote_copy(
                src_ref=bcast_ref.at[rows],
                dst_ref=ag_recv_ref.at[slot, rows],
                send_sem=p2_send.at[slot * 2 + half],
                recv_sem=p2_recv.at[slot * 2 + half],
                device_id=(peer,),
                device_id_type=pl.DeviceIdType.MESH,
            )
            rdma.start()
            p2_rdmas.append(rdma)
        _scatter_out_half(out_ref, red_b, my, half)

    for half in range(2):
        for s in (0, 2, 1):
            rows = pl.ds(half * HG, HG)
            recv = pltpu.make_async_remote_copy(
                src_ref=ag_recv_ref.at[s, rows],
                dst_ref=ag_recv_ref.at[s, rows],
                send_sem=p2_send.at[s * 2 + half],
                recv_sem=p2_recv.at[s * 2 + half],
                device_id=(my,),
                device_id_type=pl.DeviceIdType.MESH,
            )
            recv.wait_recv()
            chunk = lax.rem(my + s + 1, N_DEV)
            _scatter_out_half(out_ref, ag_recv_ref[s, half * HG:(half + 1) * HG, :],
                              chunk, half)

    for rdma in p1_rdmas + p2_rdmas:
        rdma.wait_send()


def kernel(x, Wq, K_ext, V_ext, Wo):
    out = pl.pallas_call(
        _body,
        out_shape=jax.ShapeDtypeStruct((SQ, D_MODEL), jnp.bfloat16),
        in_specs=[pl.BlockSpec(memory_space=pl.ANY)] * 5,
        out_specs=pl.BlockSpec(memory_space=pltpu.VMEM),
        scratch_shapes=[
            pltpu.VMEM((2, RG, D_MODEL), jnp.float32),
            pltpu.VMEM((D_MODEL, D_MODEL), jnp.float32),
            pltpu.VMEM((D_MODEL, D_MODEL), jnp.float32),
            pltpu.VMEM((2, RG, H_LOC, DH), jnp.float32),
            pltpu.VMEM((2, RG, H_LOC, DH), jnp.float32),
            pltpu.VMEM((RG, D_MODEL), jnp.bfloat16),
            pltpu.VMEM((RG, H_LOC, DH), jnp.bfloat16),
            pltpu.VMEM((RG, H_LOC, DH), jnp.bfloat16),
            pltpu.VMEM((RG, D_MODEL), jnp.float32),
            pltpu.VMEM((D_MODEL, D_MODEL), jnp.bfloat16),
            pltpu.VMEM((D_MODEL, D_MODEL), jnp.bfloat16),
            pltpu.VMEM((3, RG, D_MODEL), jnp.bfloat16),
            pltpu.VMEM((3, RG, D_MODEL), jnp.bfloat16),
            pltpu.VMEM((RG, D_MODEL), jnp.bfloat16),
            pltpu.VMEM((3, RG, D_MODEL), jnp.bfloat16),
            pltpu.SemaphoreType.DMA((2,)),
            pltpu.SemaphoreType.DMA,
            pltpu.SemaphoreType.DMA((6,)),
            pltpu.SemaphoreType.DMA((6,)),
            pltpu.SemaphoreType.DMA((6,)),
            pltpu.SemaphoreType.DMA((6,)),
            pltpu.VMEM((RG, H_LOC * DH), jnp.bfloat16),
        ],
        compiler_params=pltpu.CompilerParams(
            collective_id=0, vmem_limit_bytes=100 * 1024 * 1024),
    )(x[0], Wq, K_ext[0], V_ext[0], Wo)

    return out[None]
